# Initial kernel scaffold; baseline (speedup 1.0000x reference)
import jax
import jax.numpy as jnp
from jax.experimental import pallas as pl


def kernel(edges, rings):
    raise NotImplementedError("write your pallas kernel here")


# speedup vs baseline: 8.9582x; 8.9582x over previous
import functools

import jax
import jax.numpy as jnp
from jax import lax
from jax.experimental import pallas as pl
from jax.experimental.pallas import tpu as pltpu
from jax.experimental.pallas import tpu_sc as plsc

B = 2048
A = 64
D = 6
R = 16
S = 8
L = 16
NC, NS = 2, 16
NW = NC * NS
NB = B // NW
EW = A * D
EV = EW // L
RW = R * S


def _find_ring_bonds_body(edges_hbm, rings_hbm, aidx_hbm, out_hbm,
                          e_v, r_v, o_v, ai_v, rb_v):
    wid = lax.axis_index("c") * NS + lax.axis_index("s")
    base = wid * NB
    pltpu.sync_copy(edges_hbm.at[pl.ds(base, NB)], e_v)
    pltpu.sync_copy(rings_hbm.at[pl.ds(base, NB)], r_v)
    pltpu.sync_copy(aidx_hbm, ai_v)

    lane = lax.iota(jnp.int32, L)
    lo_mask = lane < S
    hi_mask = lane >= S
    zeros = jnp.zeros((L,), jnp.int32)

    def item(b, carry):
        for i in range(A // L):
            rb_v[pl.ds(i * L, L)] = zeros
        for r2 in range(0, R, 2):
            idx = r_v[b, pl.ds(r2 * S, L)]
            cur = plsc.load_gather(rb_v, [idx], mask=lo_mask)
            plsc.store_scatter(rb_v, [idx], cur | (1 << r2), mask=lo_mask)
            cur = plsc.load_gather(rb_v, [idx], mask=hi_mask)
            plsc.store_scatter(rb_v, [idx], cur | (1 << (r2 + 1)), mask=hi_mask)
        for v in range(EV):
            nbr_idx = e_v[b, pl.ds(v * L, L)]
            self_idx = ai_v[pl.ds(v * L, L)]
            nbr_bits = plsc.load_gather(rb_v, [nbr_idx])
            self_bits = plsc.load_gather(rb_v, [self_idx])
            val = jnp.where((nbr_bits & self_bits) != 0,
                            jnp.float32(1.0), jnp.float32(0.0))
            o_v[b, pl.ds(v * L, L)] = val
        return carry

    lax.fori_loop(0, NB, item, 0)
    pltpu.sync_copy(o_v, out_hbm.at[pl.ds(base, NB)])


@jax.jit
def kernel(edges, rings):
    edges_i = edges.astype(jnp.int32).reshape(B, EW)
    rings_f = rings.reshape(B, RW)
    aidx = (jnp.arange(EW, dtype=jnp.int32) // D)

    mesh = plsc.VectorSubcoreMesh(core_axis_name="c", subcore_axis_name="s",
                                  num_cores=NC, num_subcores=NS)
    run = pl.kernel(
        _find_ring_bonds_body,
        out_type=jax.ShapeDtypeStruct((B, EW), jnp.float32),
        mesh=mesh,
        scratch_types=[
            pltpu.VMEM((NB, EW), jnp.int32),
            pltpu.VMEM((NB, RW), jnp.int32),
            pltpu.VMEM((NB, EW), jnp.float32),
            pltpu.VMEM((EW,), jnp.int32),
            pltpu.VMEM((A,), jnp.int32),
        ],
        compiler_params=pltpu.CompilerParams(needs_layout_passes=False),
    )
    out = run(edges_i, rings_f, aidx)
    return out.reshape(B, A, D, 1)


# speedup vs baseline: 9.3179x; 1.0402x over previous
import functools

import jax
import jax.numpy as jnp
from jax import lax
from jax.experimental import pallas as pl
from jax.experimental.pallas import tpu as pltpu
from jax.experimental.pallas import tpu_sc as plsc

B = 2048
A = 64
D = 6
R = 16
S = 8
L = 16
NC, NS = 2, 16
NW = NC * NS
NB = B // NW
EW = A * D
EV = EW // L
RW = R * S


def _find_ring_bonds_body(edges_hbm, rings_hbm, aidx_hbm, out_hbm,
                          e_v, r_v, o_v, ai_v, rb_v, ta_v, tb_v):
    wid = lax.axis_index("c") * NS + lax.axis_index("s")
    base = wid * NB
    pltpu.sync_copy(edges_hbm.at[pl.ds(base, NB)], e_v)
    pltpu.sync_copy(rings_hbm.at[pl.ds(base, NB)], r_v)
    pltpu.sync_copy(aidx_hbm, ai_v)

    lane = lax.iota(jnp.int32, L)
    lo_mask = lane < S
    zeros = jnp.zeros((L,), jnp.int32)
    nib = jnp.full((L,), 0x11111111, jnp.int32)

    def item(b, carry):
        for i in range(A // L):
            ta_v[pl.ds(i * L, L)] = zeros
            tb_v[pl.ds(i * L, L)] = zeros
        for r2 in range(0, R, 2):
            idx = r_v[b, pl.ds(r2 * S, L)]
            rr = r2 % 8
            val = jnp.where(lo_mask, 1 << (4 * rr), 1 << (4 * (rr + 1)))
            tab = ta_v if r2 < 8 else tb_v
            plsc.addupdate_scatter(tab, [idx], val)
        for i in range(A // L):
            va = ta_v[pl.ds(i * L, L)]
            vb = tb_v[pl.ds(i * L, L)]
            va = va | lax.shift_right_logical(va, 1)
            va = (va | lax.shift_right_logical(va, 2)) & nib
            vb = vb | lax.shift_right_logical(vb, 1)
            vb = (vb | lax.shift_right_logical(vb, 2)) & nib
            rb_v[pl.ds(i * L, L)] = va | (vb << 1)
        for v in range(EV):
            nbr_idx = e_v[b, pl.ds(v * L, L)]
            self_idx = ai_v[pl.ds(v * L, L)]
            nbr_bits = plsc.load_gather(rb_v, [nbr_idx])
            self_bits = plsc.load_gather(rb_v, [self_idx])
            val = jnp.where((nbr_bits & self_bits) != 0,
                            jnp.float32(1.0), jnp.float32(0.0))
            o_v[b, pl.ds(v * L, L)] = val
        return carry

    lax.fori_loop(0, NB, item, 0)
    pltpu.sync_copy(o_v, out_hbm.at[pl.ds(base, NB)])


@jax.jit
def kernel(edges, rings):
    edges_i = edges.astype(jnp.int32).reshape(B, EW)
    rings_f = rings.reshape(B, RW)
    aidx = (jnp.arange(EW, dtype=jnp.int32) // D)

    mesh = plsc.VectorSubcoreMesh(core_axis_name="c", subcore_axis_name="s",
                                  num_cores=NC, num_subcores=NS)
    run = pl.kernel(
        _find_ring_bonds_body,
        out_type=jax.ShapeDtypeStruct((B, EW), jnp.float32),
        mesh=mesh,
        scratch_types=[
            pltpu.VMEM((NB, EW), jnp.int32),
            pltpu.VMEM((NB, RW), jnp.int32),
            pltpu.VMEM((NB, EW), jnp.float32),
            pltpu.VMEM((EW,), jnp.int32),
            pltpu.VMEM((A,), jnp.int32),
            pltpu.VMEM((A,), jnp.int32),
            pltpu.VMEM((A,), jnp.int32),
        ],
        compiler_params=pltpu.CompilerParams(needs_layout_passes=False),
    )
    out = run(edges_i, rings_f, aidx)
    return out.reshape(B, A, D, 1)
